# baseline (speedup 1.0000x reference)
import functools

import jax
import jax.numpy as jnp
from jax import lax
from jax.experimental import pallas as pl
from jax.experimental.pallas import tpu as pltpu
from jax.experimental.pallas import tpu_sc as plsc

N_NODES = 10000
D_IN = 256
D_OUT = 256
HALF = 128
NC, NS = 2, 16
CHUNK = 128
RING = 4
LANES = 16
ROWS_A = 624
ROWS_B = 640

_BCAST_DNUMS = lax.GatherDimensionNumbers(
    offset_dims=(), collapsed_slice_dims=(0,), start_index_map=(0,))


def _lane_broadcast(vec, lane):
    idx = jnp.full((LANES, 1), lane, jnp.int32)
    return lax.gather(vec, idx, _BCAST_DNUMS, (1,),
                      mode=lax.GatherScatterMode.PROMISE_IN_BOUNDS)


def _matmul_body(x_ref, w_ref, out_ref):
    res = jnp.dot(x_ref[...], w_ref[...],
                  preferred_element_type=jnp.float32)
    out_ref[0] = res[:, :HALF]
    out_ref[1] = res[:, HALF:]


def _tc_support(x, W):
    br = 1000
    return pl.pallas_call(
        _matmul_body,
        grid=(N_NODES // br,),
        in_specs=[
            pl.BlockSpec((br, D_IN), lambda i: (i, 0)),
            pl.BlockSpec((D_IN, D_OUT), lambda i: (0, 0)),
        ],
        out_specs=pl.BlockSpec((NC, br, HALF), lambda i: (0, i, 0)),
        out_shape=jax.ShapeDtypeStruct((NC, N_NODES, HALF), jnp.float32),
    )(x, W)


def _make_sc_spmm(cpt):
    mesh = plsc.VectorSubcoreMesh(core_axis_name="c", subcore_axis_name="s",
                                  num_cores=NC, num_subcores=NS)

    @functools.partial(
        pl.kernel,
        out_type=jax.ShapeDtypeStruct((N_NODES, D_OUT), jnp.float32),
        mesh=mesh,
        scratch_types=[
            pltpu.VMEM_SHARED((N_NODES, HALF), jnp.float32),
            pltpu.VMEM((CHUNK, HALF), jnp.float32),
            pltpu.VMEM((CHUNK, HALF), jnp.float32),
            pltpu.VMEM((CHUNK, HALF), jnp.float32),
            pltpu.VMEM((RING, CHUNK), jnp.int32),
            pltpu.VMEM((RING, CHUNK), jnp.int32),
            pltpu.VMEM((RING, CHUNK), jnp.float32),
            pltpu.VMEM((HALF,), jnp.float32),
            pltpu.SemaphoreType.DMA,
            pltpu.SemaphoreType.DMA,
            pltpu.SemaphoreType.DMA,
            pltpu.SemaphoreType.DMA,
            pltpu.SemaphoreType.DMA,
        ],
    )
    def sc_spmm(src_ref, dst_ref, ew_ref, b_ref, sup0_ref, sup1_ref,
                out_ref, acc, b0, b1, b2, src_g, dst_g, w_g, bbuf,
                gsem, ssem, isem_s, isem_d, isem_w):
        c = lax.axis_index("c")
        s = lax.axis_index("s")
        bufs = (b0, b1, b2)
        row0 = s * ROWS_A

        def fire_idx(k):
            slot = k & 3
            base = s * cpt + k
            pltpu.async_copy(src_ref.at[base], src_g.at[slot], isem_s)
            pltpu.async_copy(dst_ref.at[base], dst_g.at[slot], isem_d)
            pltpu.async_copy(ew_ref.at[base], w_g.at[slot], isem_w)

        def wait_idx():
            pltpu.make_async_copy(src_ref.at[0], src_g.at[0],
                                  isem_s).wait()
            pltpu.make_async_copy(dst_ref.at[0], dst_g.at[0], isem_d).wait()
            pltpu.make_async_copy(ew_ref.at[0], w_g.at[0], isem_w).wait()

        def fire_gather(slot, buf):
            @pl.when(c == 0)
            def _():
                pltpu.async_copy(sup0_ref.at[src_g.at[slot]], buf, gsem)

            @pl.when(c == 1)
            def _():
                pltpu.async_copy(sup1_ref.at[src_g.at[slot]], buf, gsem)

        def wait_gather(buf):
            pltpu.make_async_copy(sup0_ref.at[src_g.at[0]], buf, gsem).wait()

        def wait_scatter():
            pltpu.make_async_copy(b0, acc.at[dst_g.at[0]], ssem).wait()

        pltpu.sync_copy(b_ref.at[pl.ds(c * HALF, HALF)], bbuf)
        bv = [bbuf[pl.ds(c8 * LANES, LANES)] for c8 in range(HALF // LANES)]

        @pl.loop(0, CHUNK)
        def _fill(r):
            for c8 in range(HALF // LANES):
                b0[r, pl.ds(c8 * LANES, LANES)] = bv[c8]

        @pl.when(s < NS - 1)
        def _():
            for k in range(ROWS_A // CHUNK):
                pltpu.sync_copy(b0, acc.at[pl.ds(row0 + k * CHUNK, CHUNK)])
            rem = ROWS_A % CHUNK
            pltpu.sync_copy(
                b0.at[pl.ds(0, rem)],
                acc.at[pl.ds(row0 + (ROWS_A // CHUNK) * CHUNK, rem)])

        @pl.when(s == NS - 1)
        def _():
            for k in range(ROWS_B // CHUNK):
                pltpu.sync_copy(
                    b0, acc.at[pl.ds((NS - 1) * ROWS_A + k * CHUNK, CHUNK)])

        plsc.subcore_barrier()

        fire_idx(0)
        fire_idx(1)
        fire_idx(2)
        wait_idx()
        fire_gather(0, b0)
        wait_idx()
        fire_gather(1, b1)

        @pl.loop(0, cpt)
        def _step(k):
            for i in range(3):
                @pl.when(lax.rem(k, 3) == i)
                def _(i=i):
                    buf = bufs[i]
                    nbuf = bufs[(i + 2) % 3]

                    wait_gather(buf)
                    slot = k & 3

                    @pl.when(k >= 1)
                    def _():
                        wait_scatter()

                    @pl.when(k + 3 < cpt)
                    def _():
                        fire_idx(k + 3)

                    @pl.when(k + 2 < cpt)
                    def _():
                        wait_idx()
                        fire_gather((k + 2) & 3, nbuf)

                    @pl.loop(0, CHUNK // LANES)
                    def _scale(g):
                        wgrp = w_g[slot, pl.ds(g * LANES, LANES)]
                        for u in range(LANES):
                            e = g * LANES + u
                            wv = _lane_broadcast(wgrp, u)
                            for c8 in range(HALF // LANES):
                                sl = pl.ds(c8 * LANES, LANES)
                                buf[e, sl] = buf[e, sl] * wv

                    pltpu.async_copy(buf, acc.at[dst_g.at[slot]], ssem,
                                     add=True)

        wait_scatter()
        plsc.subcore_barrier()

        @pl.when(s < NS - 1)
        def _():
            pltpu.sync_copy(
                acc.at[pl.ds(row0, ROWS_A)],
                out_ref.at[pl.ds(row0, ROWS_A), pl.ds(c * HALF, HALF)])

        @pl.when(s == NS - 1)
        def _():
            pltpu.sync_copy(
                acc.at[pl.ds((NS - 1) * ROWS_A, ROWS_B)],
                out_ref.at[pl.ds((NS - 1) * ROWS_A, ROWS_B),
                           pl.ds(c * HALF, HALF)])

    return sc_spmm


def kernel(x, edge_index, edge_weight, W, b):
    support = _tc_support(x, W)

    e = edge_index.shape[1]
    cpt = -(-e // (NS * CHUNK))
    e_pad = NS * cpt * CHUNK
    ei = jnp.pad(edge_index, ((0, 0), (0, e_pad - e)))
    ew = jnp.pad(edge_weight, (0, e_pad - e)).reshape(NS * cpt, CHUNK)
    src = ei[0].reshape(NS * cpt, CHUNK)
    dst = ei[1].reshape(NS * cpt, CHUNK)

    return _make_sc_spmm(cpt)(src, dst, ew, b, support[0], support[1])


# speedup vs baseline: 1.0141x; 1.0141x over previous
import functools

import jax
import jax.numpy as jnp
from jax import lax
from jax.experimental import pallas as pl
from jax.experimental.pallas import tpu as pltpu
from jax.experimental.pallas import tpu_sc as plsc

N_NODES = 10000
D_IN = 256
D_OUT = 256
HALF = 128
NC, NS = 2, 16
CHUNK = 128
RING = 4
LANES = 16
ROWS_A = 624
ROWS_B = 640

_BCAST_DNUMS = lax.GatherDimensionNumbers(
    offset_dims=(), collapsed_slice_dims=(0,), start_index_map=(0,))


def _lane_broadcast(vec, lane):
    idx = jnp.full((LANES, 1), lane, jnp.int32)
    return lax.gather(vec, idx, _BCAST_DNUMS, (1,),
                      mode=lax.GatherScatterMode.PROMISE_IN_BOUNDS)


def _matmul_body(x_ref, w_ref, out_ref):
    res = jnp.dot(x_ref[...], w_ref[...],
                  preferred_element_type=jnp.float32)
    out_ref[0] = res[:, :HALF]
    out_ref[1] = res[:, HALF:]


def _tc_support(x, W):
    br = 1000
    return pl.pallas_call(
        _matmul_body,
        grid=(N_NODES // br,),
        in_specs=[
            pl.BlockSpec((br, D_IN), lambda i: (i, 0)),
            pl.BlockSpec((D_IN, D_OUT), lambda i: (0, 0)),
        ],
        out_specs=pl.BlockSpec((NC, br, HALF), lambda i: (0, i, 0)),
        out_shape=jax.ShapeDtypeStruct((NC, N_NODES, HALF), jnp.float32),
    )(x, W)


def _make_sc_spmm(cpt):
    mesh = plsc.VectorSubcoreMesh(core_axis_name="c", subcore_axis_name="s",
                                  num_cores=NC, num_subcores=NS)

    @functools.partial(
        pl.kernel,
        out_type=jax.ShapeDtypeStruct((N_NODES, D_OUT), jnp.float32),
        mesh=mesh,
        scratch_types=[
            pltpu.VMEM_SHARED((N_NODES, HALF), jnp.float32),
            pltpu.VMEM((CHUNK, HALF), jnp.float32),
            pltpu.VMEM((CHUNK, HALF), jnp.float32),
            pltpu.VMEM((CHUNK, HALF), jnp.float32),
            pltpu.VMEM((RING, CHUNK), jnp.int32),
            pltpu.VMEM((RING, CHUNK), jnp.int32),
            pltpu.VMEM((RING, CHUNK), jnp.float32),
            pltpu.VMEM((HALF,), jnp.float32),
            pltpu.SemaphoreType.DMA,
            pltpu.SemaphoreType.DMA,
            pltpu.SemaphoreType.DMA,
            pltpu.SemaphoreType.DMA,
            pltpu.SemaphoreType.DMA,
        ],
    )
    def sc_spmm(src_ref, dst_ref, ew_ref, b_ref, sup0_ref, sup1_ref,
                out_ref, acc, b0, b1, b2, src_g, dst_g, w_g, bbuf,
                gsem, ssem, isem_s, isem_d, isem_w):
        c = lax.axis_index("c")
        s = lax.axis_index("s")
        bufs = (b0, b1, b2)
        row0 = s * ROWS_A

        def fire_idx(k):
            slot = k & 3
            base = s * cpt + k
            pltpu.async_copy(src_ref.at[base], src_g.at[slot], isem_s)
            pltpu.async_copy(dst_ref.at[base], dst_g.at[slot], isem_d)
            pltpu.async_copy(ew_ref.at[base], w_g.at[slot], isem_w)

        def wait_idx():
            pltpu.make_async_copy(src_ref.at[0], src_g.at[0],
                                  isem_s).wait()
            pltpu.make_async_copy(dst_ref.at[0], dst_g.at[0], isem_d).wait()
            pltpu.make_async_copy(ew_ref.at[0], w_g.at[0], isem_w).wait()

        def fire_gather(slot, buf):
            @pl.when(c == 0)
            def _():
                pltpu.async_copy(sup0_ref.at[src_g.at[slot]], buf, gsem)

            @pl.when(c == 1)
            def _():
                pltpu.async_copy(sup1_ref.at[src_g.at[slot]], buf, gsem)

        def wait_gather(buf):
            pltpu.make_async_copy(sup0_ref.at[src_g.at[0]], buf, gsem).wait()

        def wait_scatter():
            pltpu.make_async_copy(b0, acc.at[dst_g.at[0]], ssem).wait()

        pltpu.sync_copy(b_ref.at[pl.ds(c * HALF, HALF)], bbuf)
        bv = [bbuf[pl.ds(c8 * LANES, LANES)] for c8 in range(HALF // LANES)]

        @pl.loop(0, CHUNK)
        def _fill(r):
            for c8 in range(HALF // LANES):
                b0[r, pl.ds(c8 * LANES, LANES)] = bv[c8]

        @pl.when(s < NS - 1)
        def _():
            for k in range(ROWS_A // CHUNK):
                pltpu.sync_copy(b0, acc.at[pl.ds(row0 + k * CHUNK, CHUNK)])
            rem = ROWS_A % CHUNK
            pltpu.sync_copy(
                b0.at[pl.ds(0, rem)],
                acc.at[pl.ds(row0 + (ROWS_A // CHUNK) * CHUNK, rem)])

        @pl.when(s == NS - 1)
        def _():
            for k in range(ROWS_B // CHUNK):
                pltpu.sync_copy(
                    b0, acc.at[pl.ds((NS - 1) * ROWS_A + k * CHUNK, CHUNK)])

        plsc.subcore_barrier()

        fire_idx(0)
        fire_idx(1)
        fire_idx(2)
        wait_idx()
        fire_gather(0, b0)
        wait_idx()
        fire_gather(1, b1)

        @pl.loop(0, cpt)
        def _step(k):
            for i in range(3):
                @pl.when(lax.rem(k, 3) == i)
                def _(i=i):
                    buf = bufs[i]
                    nbuf = bufs[(i + 2) % 3]

                    wait_gather(buf)
                    slot = k & 3

                    @pl.loop(0, CHUNK // LANES)
                    def _scale(g):
                        wgrp = w_g[slot, pl.ds(g * LANES, LANES)]
                        for u in range(LANES):
                            e = g * LANES + u
                            wv = _lane_broadcast(wgrp, u)
                            for c8 in range(HALF // LANES):
                                sl = pl.ds(c8 * LANES, LANES)
                                buf[e, sl] = buf[e, sl] * wv

                    @pl.when(k >= 1)
                    def _():
                        wait_scatter()

                    @pl.when(k + 3 < cpt)
                    def _():
                        fire_idx(k + 3)

                    @pl.when(k + 2 < cpt)
                    def _():
                        wait_idx()
                        fire_gather((k + 2) & 3, nbuf)

                    pltpu.async_copy(buf, acc.at[dst_g.at[slot]], ssem,
                                     add=True)

        wait_scatter()
        plsc.subcore_barrier()

        @pl.when(s < NS - 1)
        def _():
            pltpu.sync_copy(
                acc.at[pl.ds(row0, ROWS_A)],
                out_ref.at[pl.ds(row0, ROWS_A), pl.ds(c * HALF, HALF)])

        @pl.when(s == NS - 1)
        def _():
            pltpu.sync_copy(
                acc.at[pl.ds((NS - 1) * ROWS_A, ROWS_B)],
                out_ref.at[pl.ds((NS - 1) * ROWS_A, ROWS_B),
                           pl.ds(c * HALF, HALF)])

    return sc_spmm


def kernel(x, edge_index, edge_weight, W, b):
    support = _tc_support(x, W)

    e = edge_index.shape[1]
    cpt = -(-e // (NS * CHUNK))
    e_pad = NS * cpt * CHUNK
    ei = jnp.pad(edge_index, ((0, 0), (0, e_pad - e)))
    ew = jnp.pad(edge_weight, (0, e_pad - e)).reshape(NS * cpt, CHUNK)
    src = ei[0].reshape(NS * cpt, CHUNK)
    dst = ei[1].reshape(NS * cpt, CHUNK)

    return _make_sc_spmm(cpt)(src, dst, ew, b, support[0], support[1])


# speedup vs baseline: 1.0684x; 1.0535x over previous
import functools

import jax
import jax.numpy as jnp
from jax import lax
from jax.experimental import pallas as pl
from jax.experimental.pallas import tpu as pltpu
from jax.experimental.pallas import tpu_sc as plsc

N_NODES = 10000
D_IN = 256
D_OUT = 256
HALF = 128
NC, NS = 2, 16
CHUNK = 128
RING = 4
LANES = 16
ROWS_A = 624
ROWS_B = 640

_BCAST_DNUMS = lax.GatherDimensionNumbers(
    offset_dims=(), collapsed_slice_dims=(0,), start_index_map=(0,))


def _lane_broadcast(vec, lane):
    idx = jnp.full((LANES, 1), lane, jnp.int32)
    return lax.gather(vec, idx, _BCAST_DNUMS, (1,),
                      mode=lax.GatherScatterMode.PROMISE_IN_BOUNDS)


def _matmul_body(x_ref, w_ref, out_ref):
    res = jnp.dot(x_ref[...], w_ref[...],
                  preferred_element_type=jnp.float32)
    out_ref[0] = res[:, :HALF]
    out_ref[1] = res[:, HALF:]


def _tc_support(x, W):
    br = 1000
    return pl.pallas_call(
        _matmul_body,
        grid=(N_NODES // br,),
        in_specs=[
            pl.BlockSpec((br, D_IN), lambda i: (i, 0)),
            pl.BlockSpec((D_IN, D_OUT), lambda i: (0, 0)),
        ],
        out_specs=pl.BlockSpec((NC, br, HALF), lambda i: (0, i, 0)),
        out_shape=jax.ShapeDtypeStruct((NC, N_NODES, HALF), jnp.float32),
    )(x, W)


def _make_sc_spmm(cpt):
    mesh = plsc.VectorSubcoreMesh(core_axis_name="c", subcore_axis_name="s",
                                  num_cores=NC, num_subcores=NS)

    @functools.partial(
        pl.kernel,
        out_type=jax.ShapeDtypeStruct((N_NODES, D_OUT), jnp.float32),
        mesh=mesh,
        scratch_types=[
            pltpu.VMEM_SHARED((N_NODES, HALF), jnp.float32),
            pltpu.VMEM((CHUNK, HALF), jnp.float32),
            pltpu.VMEM((CHUNK, HALF), jnp.float32),
            pltpu.VMEM((CHUNK, HALF), jnp.float32),
            pltpu.VMEM((RING, CHUNK), jnp.int32),
            pltpu.VMEM((RING, CHUNK), jnp.int32),
            pltpu.VMEM((RING, CHUNK), jnp.float32),
            pltpu.VMEM((HALF,), jnp.float32),
            pltpu.SemaphoreType.DMA,
            pltpu.SemaphoreType.DMA,
            pltpu.SemaphoreType.DMA,
            pltpu.SemaphoreType.DMA,
            pltpu.SemaphoreType.DMA,
        ],
    )
    def sc_spmm(src_ref, dst_ref, ew_ref, b_ref, sup0_ref, sup1_ref,
                out_ref, acc, b0, b1, b2, src_g, dst_g, w_g, bbuf,
                gsem, ssem, isem_s, isem_d, isem_w):
        c = lax.axis_index("c")
        s = lax.axis_index("s")
        bufs = (b0, b1, b2)
        row0 = s * ROWS_A

        def fire_idx(k):
            slot = k & 3
            base = s * cpt + k
            pltpu.async_copy(src_ref.at[base], src_g.at[slot], isem_s)
            pltpu.async_copy(dst_ref.at[base], dst_g.at[slot], isem_d)
            pltpu.async_copy(ew_ref.at[base], w_g.at[slot], isem_w)

        def wait_idx():
            pltpu.make_async_copy(src_ref.at[0], src_g.at[0],
                                  isem_s).wait()
            pltpu.make_async_copy(dst_ref.at[0], dst_g.at[0], isem_d).wait()
            pltpu.make_async_copy(ew_ref.at[0], w_g.at[0], isem_w).wait()

        def fire_gather(slot, buf):
            @pl.when(c == 0)
            def _():
                pltpu.async_copy(sup0_ref.at[src_g.at[slot]], buf, gsem)

            @pl.when(c == 1)
            def _():
                pltpu.async_copy(sup1_ref.at[src_g.at[slot]], buf, gsem)

        def wait_gather(buf):
            pltpu.make_async_copy(sup0_ref.at[src_g.at[0]], buf, gsem).wait()

        def wait_scatter():
            pltpu.make_async_copy(b0, acc.at[dst_g.at[0]], ssem).wait()

        pltpu.sync_copy(b_ref.at[pl.ds(c * HALF, HALF)], bbuf)
        bv = [bbuf[pl.ds(c8 * LANES, LANES)] for c8 in range(HALF // LANES)]

        @pl.loop(0, CHUNK)
        def _fill(r):
            for c8 in range(HALF // LANES):
                b0[r, pl.ds(c8 * LANES, LANES)] = bv[c8]

        @pl.when(s < NS - 1)
        def _():
            for k in range(ROWS_A // CHUNK):
                pltpu.sync_copy(b0, acc.at[pl.ds(row0 + k * CHUNK, CHUNK)])
            rem = ROWS_A % CHUNK
            pltpu.sync_copy(
                b0.at[pl.ds(0, rem)],
                acc.at[pl.ds(row0 + (ROWS_A // CHUNK) * CHUNK, rem)])

        @pl.when(s == NS - 1)
        def _():
            for k in range(ROWS_B // CHUNK):
                pltpu.sync_copy(
                    b0, acc.at[pl.ds((NS - 1) * ROWS_A + k * CHUNK, CHUNK)])

        plsc.subcore_barrier()

        fire_idx(0)
        fire_idx(1)
        fire_idx(2)
        wait_idx()
        fire_gather(0, b0)
        wait_idx()
        fire_gather(1, b1)

        @pl.loop(0, cpt)
        def _step(k):
            for i in range(3):
                @pl.when(lax.rem(k, 3) == i)
                def _(i=i):
                    buf = bufs[i]
                    nbuf = bufs[(i + 2) % 3]

                    wait_gather(buf)
                    slot = k & 3

                    @pl.loop(0, CHUNK // LANES)
                    def _scale(g):
                        wgrp = w_g[slot, pl.ds(g * LANES, LANES)]
                        for u in range(LANES):
                            e = g * LANES + u
                            wv = _lane_broadcast(wgrp, u)
                            for c8 in range(HALF // LANES):
                                sl = pl.ds(c8 * LANES, LANES)
                                buf[e, sl] = buf[e, sl] * wv

                    @pl.when(k >= 1)
                    def _():
                        wait_scatter()

                    @pl.when(k + 3 < cpt)
                    def _():
                        fire_idx(k + 3)

                    @pl.when(k + 2 < cpt)
                    def _():
                        wait_idx()
                        fire_gather((k + 2) & 3, nbuf)

                    pltpu.async_copy(buf, acc.at[dst_g.at[slot]], ssem,
                                     add=True)

        wait_scatter()
        plsc.subcore_barrier()

        @pl.when(s < NS - 1)
        def _():
            pltpu.sync_copy(
                acc.at[pl.ds(row0, ROWS_A)],
                out_ref.at[pl.ds(row0, ROWS_A), pl.ds(c * HALF, HALF)])

        @pl.when(s == NS - 1)
        def _():
            pltpu.sync_copy(
                acc.at[pl.ds((NS - 1) * ROWS_A, ROWS_B)],
                out_ref.at[pl.ds((NS - 1) * ROWS_A, ROWS_B),
                           pl.ds(c * HALF, HALF)])

    return sc_spmm


def _make_prep(full, rows):
    padr = rows - full

    def _prep_body(ei_ref, ew_ref, src_ref, dst_ref, w_ref):
        zi = jnp.zeros((padr, CHUNK), jnp.int32)
        zf = jnp.zeros((padr, CHUNK), jnp.float32)
        src_ref[...] = jnp.concatenate([ei_ref[0], zi], axis=0)
        dst_ref[...] = jnp.concatenate([ei_ref[1], zi], axis=0)
        w_ref[...] = jnp.concatenate([ew_ref[...], zf], axis=0)

    return pl.pallas_call(
        _prep_body,
        out_shape=[jax.ShapeDtypeStruct((rows, CHUNK), jnp.int32),
                   jax.ShapeDtypeStruct((rows, CHUNK), jnp.int32),
                   jax.ShapeDtypeStruct((rows, CHUNK), jnp.float32)])


def kernel(x, edge_index, edge_weight, W, b):
    support = _tc_support(x, W)

    e = edge_index.shape[1]
    cpt = -(-e // (NS * CHUNK))
    rows = NS * cpt
    if e % CHUNK == 0:
        src, dst, ew = _make_prep(e // CHUNK, rows)(
            edge_index.reshape(2, e // CHUNK, CHUNK),
            edge_weight.reshape(e // CHUNK, CHUNK))
    else:
        e_pad = rows * CHUNK
        ei = jnp.pad(edge_index, ((0, 0), (0, e_pad - e)))
        ew = jnp.pad(edge_weight, (0, e_pad - e)).reshape(rows, CHUNK)
        src = ei[0].reshape(rows, CHUNK)
        dst = ei[1].reshape(rows, CHUNK)

    return _make_sc_spmm(cpt)(src, dst, ew, b, support[0], support[1])
